# baseline (speedup 1.0000x reference)
import functools

import jax
import jax.numpy as jnp
from jax import lax
from jax.experimental import pallas as pl
from jax.experimental.pallas import tpu as pltpu
from jax.experimental.pallas import tpu_sc as plsc

N = 50000
E = 800000
INP = 32
EMB = 32
NB = 4
NR = 64
OUT = 32
H4W = NB * OUT

NC = 2
NS = 16
CH = 128
HW = NB * 16
MW = 16
NCH = 392
EP = NS * NCH * CH
SUP = 28
NSUP = NCH // SUP
NPAIR = SUP // 2
STRIPE = N // NS
ZR = STRIPE // 5


def _edge_body(h4a, h4b, eidx, comp, out, ibuf, rows0, rows1, cf0, cf1,
               msg0, msg1, zbuf, agg, gsem0, gsem1, csem0, csem1,
               ssem0, ssem1):
    cid = lax.axis_index("c")
    sid = lax.axis_index("s")
    r0 = sid * STRIPE

    zeros16 = jnp.zeros((16,), jnp.float32)

    def zb(i, carry):
        zbuf[i, pl.ds(0, 16)] = zeros16
        return carry

    lax.fori_loop(0, ZR, zb, 0)
    for j in range(STRIPE // ZR):
        pltpu.sync_copy(zbuf, agg.at[pl.ds(r0 + j * ZR, ZR)])
    plsc.subcore_barrier()

    def make_compute(rows_b, cf_b, msg_b):
        def compute(k):
            del k

            def group(g, carry):
                e0 = g * 16
                for j in range(16):
                    e = e0 + j
                    m = (cf_b[e, pl.ds(0, 16)] * rows_b[e, pl.ds(0, 16)]
                         + cf_b[e, pl.ds(16, 16)] * rows_b[e, pl.ds(16, 16)]
                         + cf_b[e, pl.ds(32, 16)] * rows_b[e, pl.ds(32, 16)]
                         + cf_b[e, pl.ds(48, 16)] * rows_b[e, pl.ds(48, 16)])
                    msg_b[e, pl.ds(0, 16)] = m
                return carry

            lax.fori_loop(0, CH // 16, group, 0)
        return compute

    def run_edges(h4):
        compute0 = make_compute(rows0, cf0, msg0)
        compute1 = make_compute(rows1, cf1, msg1)

        def g_start(k, rows_b, cf_b, gsem, csem):
            pltpu.async_copy(h4.at[ibuf.at[k, 0]], rows_b, gsem)
            pltpu.async_copy(comp.at[ibuf.at[k, 2]], cf_b, csem)

        def g_wait(rows_b, cf_b, gsem, csem):
            pltpu.make_async_copy(h4.at[ibuf.at[0, 0]], rows_b, gsem).wait()
            pltpu.make_async_copy(comp.at[ibuf.at[0, 2]], cf_b, csem).wait()

        def s_start(k, msg_b, sem):
            pltpu.async_copy(msg_b, agg.at[ibuf.at[k, 1]], sem, add=True)

        def s_wait(msg_b, sem):
            pltpu.make_async_copy(msg_b, agg.at[ibuf.at[0, 1]], sem).wait()

        def pair(j, carry):
            c0 = 2 * j
            g_wait(rows0, cf0, gsem0, csem0)

            @pl.when(j >= 1)
            def _():
                s_wait(msg0, ssem0)

            g_start(c0 + 1, rows1, cf1, gsem1, csem1)
            compute0(c0)
            s_start(c0, msg0, ssem0)

            g_wait(rows1, cf1, gsem1, csem1)

            @pl.when(j >= 1)
            def _():
                s_wait(msg1, ssem1)

            @pl.when(j < NPAIR - 1)
            def _():
                g_start(c0 + 2, rows0, cf0, gsem0, csem0)

            compute1(c0 + 1)
            s_start(c0 + 1, msg1, ssem1)
            return carry

        def super_body(s, carry):
            pltpu.sync_copy(eidx.at[sid, pl.ds(s * SUP, SUP)], ibuf)
            g_start(0, rows0, cf0, gsem0, csem0)
            lax.fori_loop(0, NPAIR, pair, 0)
            s_wait(msg0, ssem0)
            s_wait(msg1, ssem1)
            return carry

        lax.fori_loop(0, NSUP, super_body, 0)

    @pl.when(cid == 0)
    def _edges_a():
        run_edges(h4a)

    @pl.when(cid == 1)
    def _edges_b():
        run_edges(h4b)

    plsc.subcore_barrier()
    pltpu.sync_copy(agg.at[pl.ds(r0, STRIPE)], out.at[cid, pl.ds(r0, STRIPE)])


_edge_pass = functools.partial(
    pl.kernel,
    out_type=jax.ShapeDtypeStruct((NC, N, MW), jnp.float32),
    mesh=plsc.VectorSubcoreMesh(core_axis_name="c", subcore_axis_name="s"),
    compiler_params=pltpu.CompilerParams(use_tc_tiling_on_sc=False),
    scratch_types=[
        pltpu.VMEM((SUP, 3, CH), jnp.int32),
        pltpu.VMEM((CH, HW), jnp.float32),
        pltpu.VMEM((CH, HW), jnp.float32),
        pltpu.VMEM((CH, HW), jnp.float32),
        pltpu.VMEM((CH, HW), jnp.float32),
        pltpu.VMEM((CH, MW), jnp.float32),
        pltpu.VMEM((CH, MW), jnp.float32),
        pltpu.VMEM((ZR, MW), jnp.float32),
        pltpu.VMEM_SHARED((N + 16, MW), jnp.float32),
        pltpu.SemaphoreType.DMA,
        pltpu.SemaphoreType.DMA,
        pltpu.SemaphoreType.DMA,
        pltpu.SemaphoreType.DMA,
        pltpu.SemaphoreType.DMA,
        pltpu.SemaphoreType.DMA,
    ],
)(_edge_body)


BN = 1000


def _dense_in_body(x_ref, emb_ref, w_ref, h4a_ref, h4b_ref, hs_ref):
    h = jnp.concatenate([x_ref[...], emb_ref[...]], axis=1)
    o = jnp.dot(h, w_ref[...], preferred_element_type=jnp.float32)
    h4a_ref[...] = o[:, :HW]
    h4b_ref[...] = o[:, HW:2 * HW]
    hs_ref[...] = o[:, 2 * HW:]


def _fuse_mid_body(parts_ref, hs_ref, w_ref, h4a_ref, h4b_ref, hs1_ref):
    agg = jnp.concatenate([parts_ref[0], parts_ref[1]], axis=1)
    h = jnp.maximum(agg + hs_ref[...], 0.0)
    o = jnp.dot(h, w_ref[...], preferred_element_type=jnp.float32)
    h4a_ref[...] = o[:, :HW]
    h4b_ref[...] = o[:, HW:2 * HW]
    hs1_ref[...] = o[:, 2 * HW:]


def _fuse_out_body(parts_ref, hs_ref, out_ref):
    agg = jnp.concatenate([parts_ref[0], parts_ref[1]], axis=1)
    out_ref[...] = jnp.maximum(agg + hs_ref[...], 0.0)


def _row_spec(width):
    return pl.BlockSpec((BN, width), lambda i: (i, 0))


def _parts_spec():
    return pl.BlockSpec((NC, BN, MW), lambda i: (0, i, 0))


def _full_spec(shape):
    return pl.BlockSpec(shape, lambda i: tuple(0 for _ in shape))


def kernel(x, edge_index, edge_type, embed, basis0, comp0, Wself0,
           basis1, comp1, Wself1):
    in0 = INP + EMB

    pad = EP - E
    srcp = jnp.concatenate([edge_index[0], jnp.zeros((pad,), jnp.int32)])
    dstp = jnp.concatenate([edge_index[1], jnp.full((pad,), N, jnp.int32)])
    etp = jnp.concatenate([edge_type, jnp.zeros((pad,), jnp.int32)])
    eidx = (jnp.stack([srcp, dstp, etp], axis=0)
            .reshape(3, NS, NCH, CH).transpose(1, 2, 0, 3))

    def _wcat(basis, wself, kdim):
        wa = basis[:, :, :MW].transpose(1, 0, 2).reshape(kdim, HW)
        wb = basis[:, :, MW:].transpose(1, 0, 2).reshape(kdim, HW)
        return jnp.concatenate([wa, wb, wself], axis=1)

    W0 = _wcat(basis0, Wself0, in0)
    W1 = _wcat(basis1, Wself1, EMB)

    grid = (N // BN,)
    h4_shapes = [jax.ShapeDtypeStruct((N, HW), jnp.float32),
                 jax.ShapeDtypeStruct((N, HW), jnp.float32),
                 jax.ShapeDtypeStruct((N, OUT), jnp.float32)]
    h4_specs = [_row_spec(HW), _row_spec(HW), _row_spec(OUT)]

    h4a0, h4b0, hs_0 = pl.pallas_call(
        _dense_in_body,
        grid=grid,
        in_specs=[_row_spec(INP), _row_spec(EMB), _full_spec((in0, 2 * HW + OUT))],
        out_specs=h4_specs,
        out_shape=h4_shapes,
    )(x, embed, W0)

    comp0p = jnp.repeat(comp0, MW, axis=1)
    comp1p = jnp.repeat(comp1, MW, axis=1)

    parts0 = _edge_pass(h4a0, h4b0, eidx, comp0p)

    h4a1, h4b1, hs_1 = pl.pallas_call(
        _fuse_mid_body,
        grid=grid,
        in_specs=[_parts_spec(), _row_spec(OUT), _full_spec((EMB, 2 * HW + OUT))],
        out_specs=h4_specs,
        out_shape=h4_shapes,
    )(parts0, hs_0, W1)

    parts1 = _edge_pass(h4a1, h4b1, eidx, comp1p)

    out = pl.pallas_call(
        _fuse_out_body,
        grid=grid,
        in_specs=[_parts_spec(), _row_spec(OUT)],
        out_specs=_row_spec(OUT),
        out_shape=jax.ShapeDtypeStruct((N, OUT), jnp.float32),
    )(parts1, hs_1)

    return out


# speedup vs baseline: 1.8590x; 1.8590x over previous
import functools

import jax
import jax.numpy as jnp
from jax import lax
from jax.experimental import pallas as pl
from jax.experimental.pallas import tpu as pltpu
from jax.experimental.pallas import tpu_sc as plsc

N = 50000
E = 800000
INP = 32
EMB = 32
NB = 4
NR = 64
OUT = 32
H4W = NB * OUT

NC = 2
NS = 16
CH = 128
HW = NB * 16
MW = 16
NCH = 392
EP = NS * NCH * CH
SUP = 28
NSUP = NCH // SUP
NPAIR = SUP // 2
STRIPE = N // NS
ZR = STRIPE // 5


def _edge_body(h4a, h4b, eidx, comp, out, ibuf, rows0, rows1, cf0, cf1,
               msg0, msg1, zbuf, agg, compS, gsem0, gsem1, csem0, csem1,
               ssem0, ssem1):
    cid = lax.axis_index("c")
    sid = lax.axis_index("s")
    r0 = sid * STRIPE

    @pl.when(sid == 0)
    def _stage_comp():
        pltpu.sync_copy(comp, compS)

    zeros16 = jnp.zeros((16,), jnp.float32)

    def zb(i, carry):
        zbuf[i, pl.ds(0, 16)] = zeros16
        return carry

    lax.fori_loop(0, ZR, zb, 0)
    for j in range(STRIPE // ZR):
        pltpu.sync_copy(zbuf, agg.at[pl.ds(r0 + j * ZR, ZR)])
    plsc.subcore_barrier()

    def make_compute(rows_b, cf_b, msg_b):
        def compute(k):
            del k

            def group(g, carry):
                e0 = g * 16
                for j in range(16):
                    e = e0 + j
                    m = (cf_b[e, pl.ds(0, 16)] * rows_b[e, pl.ds(0, 16)]
                         + cf_b[e, pl.ds(16, 16)] * rows_b[e, pl.ds(16, 16)]
                         + cf_b[e, pl.ds(32, 16)] * rows_b[e, pl.ds(32, 16)]
                         + cf_b[e, pl.ds(48, 16)] * rows_b[e, pl.ds(48, 16)])
                    msg_b[e, pl.ds(0, 16)] = m
                return carry

            lax.fori_loop(0, CH // 16, group, 0)
        return compute

    def run_edges(h4):
        compute0 = make_compute(rows0, cf0, msg0)
        compute1 = make_compute(rows1, cf1, msg1)

        def g_start(k, rows_b, cf_b, gsem, csem):
            pltpu.async_copy(h4.at[ibuf.at[k, 0]], rows_b, gsem)
            pltpu.async_copy(compS.at[ibuf.at[k, 2]], cf_b, csem)

        def g_wait(rows_b, cf_b, gsem, csem):
            pltpu.make_async_copy(h4.at[ibuf.at[0, 0]], rows_b, gsem).wait()
            pltpu.make_async_copy(compS.at[ibuf.at[0, 2]], cf_b, csem).wait()

        def s_start(k, msg_b, sem):
            pltpu.async_copy(msg_b, agg.at[ibuf.at[k, 1]], sem, add=True)

        def s_wait(msg_b, sem):
            pltpu.make_async_copy(msg_b, agg.at[ibuf.at[0, 1]], sem).wait()

        def pair(j, carry):
            c0 = 2 * j
            g_wait(rows0, cf0, gsem0, csem0)

            @pl.when(j >= 1)
            def _():
                s_wait(msg0, ssem0)

            g_start(c0 + 1, rows1, cf1, gsem1, csem1)
            compute0(c0)
            s_start(c0, msg0, ssem0)

            g_wait(rows1, cf1, gsem1, csem1)

            @pl.when(j >= 1)
            def _():
                s_wait(msg1, ssem1)

            @pl.when(j < NPAIR - 1)
            def _():
                g_start(c0 + 2, rows0, cf0, gsem0, csem0)

            compute1(c0 + 1)
            s_start(c0 + 1, msg1, ssem1)
            return carry

        def super_body(s, carry):
            pltpu.sync_copy(eidx.at[sid, pl.ds(s * SUP, SUP)], ibuf)
            g_start(0, rows0, cf0, gsem0, csem0)
            lax.fori_loop(0, NPAIR, pair, 0)
            s_wait(msg0, ssem0)
            s_wait(msg1, ssem1)
            return carry

        lax.fori_loop(0, NSUP, super_body, 0)

    @pl.when(cid == 0)
    def _edges_a():
        run_edges(h4a)

    @pl.when(cid == 1)
    def _edges_b():
        run_edges(h4b)

    plsc.subcore_barrier()
    pltpu.sync_copy(agg.at[pl.ds(r0, STRIPE)], out.at[cid, pl.ds(r0, STRIPE)])


_edge_pass = functools.partial(
    pl.kernel,
    out_type=jax.ShapeDtypeStruct((NC, N, MW), jnp.float32),
    mesh=plsc.VectorSubcoreMesh(core_axis_name="c", subcore_axis_name="s"),
    compiler_params=pltpu.CompilerParams(use_tc_tiling_on_sc=False),
    scratch_types=[
        pltpu.VMEM((SUP, 3, CH), jnp.int32),
        pltpu.VMEM((CH, HW), jnp.float32),
        pltpu.VMEM((CH, HW), jnp.float32),
        pltpu.VMEM((CH, HW), jnp.float32),
        pltpu.VMEM((CH, HW), jnp.float32),
        pltpu.VMEM((CH, MW), jnp.float32),
        pltpu.VMEM((CH, MW), jnp.float32),
        pltpu.VMEM((ZR, MW), jnp.float32),
        pltpu.VMEM_SHARED((N + 16, MW), jnp.float32),
        pltpu.VMEM_SHARED((NR, HW), jnp.float32),
        pltpu.SemaphoreType.DMA,
        pltpu.SemaphoreType.DMA,
        pltpu.SemaphoreType.DMA,
        pltpu.SemaphoreType.DMA,
        pltpu.SemaphoreType.DMA,
        pltpu.SemaphoreType.DMA,
    ],
)(_edge_body)


BN = 1000


def _dense_in_body(x_ref, emb_ref, w_ref, h4a_ref, h4b_ref, hs_ref):
    h = jnp.concatenate([x_ref[...], emb_ref[...]], axis=1)
    o = jnp.dot(h, w_ref[...], preferred_element_type=jnp.float32)
    h4a_ref[...] = o[:, :HW]
    h4b_ref[...] = o[:, HW:2 * HW]
    hs_ref[...] = o[:, 2 * HW:]


def _fuse_mid_body(parts_ref, hs_ref, w_ref, h4a_ref, h4b_ref, hs1_ref):
    agg = jnp.concatenate([parts_ref[0], parts_ref[1]], axis=1)
    h = jnp.maximum(agg + hs_ref[...], 0.0)
    o = jnp.dot(h, w_ref[...], preferred_element_type=jnp.float32)
    h4a_ref[...] = o[:, :HW]
    h4b_ref[...] = o[:, HW:2 * HW]
    hs1_ref[...] = o[:, 2 * HW:]


def _fuse_out_body(parts_ref, hs_ref, out_ref):
    agg = jnp.concatenate([parts_ref[0], parts_ref[1]], axis=1)
    out_ref[...] = jnp.maximum(agg + hs_ref[...], 0.0)


def _row_spec(width):
    return pl.BlockSpec((BN, width), lambda i: (i, 0))


def _parts_spec():
    return pl.BlockSpec((NC, BN, MW), lambda i: (0, i, 0))


def _full_spec(shape):
    return pl.BlockSpec(shape, lambda i: tuple(0 for _ in shape))


def kernel(x, edge_index, edge_type, embed, basis0, comp0, Wself0,
           basis1, comp1, Wself1):
    in0 = INP + EMB

    pad = EP - E
    srcp = jnp.concatenate([edge_index[0], jnp.zeros((pad,), jnp.int32)])
    dstp = jnp.concatenate([edge_index[1], jnp.full((pad,), N, jnp.int32)])
    etp = jnp.concatenate([edge_type, jnp.zeros((pad,), jnp.int32)])
    eidx = (jnp.stack([srcp, dstp, etp], axis=0)
            .reshape(3, NS, NCH, CH).transpose(1, 2, 0, 3))

    def _wcat(basis, wself, kdim):
        wa = basis[:, :, :MW].transpose(1, 0, 2).reshape(kdim, HW)
        wb = basis[:, :, MW:].transpose(1, 0, 2).reshape(kdim, HW)
        return jnp.concatenate([wa, wb, wself], axis=1)

    W0 = _wcat(basis0, Wself0, in0)
    W1 = _wcat(basis1, Wself1, EMB)

    grid = (N // BN,)
    h4_shapes = [jax.ShapeDtypeStruct((N, HW), jnp.float32),
                 jax.ShapeDtypeStruct((N, HW), jnp.float32),
                 jax.ShapeDtypeStruct((N, OUT), jnp.float32)]
    h4_specs = [_row_spec(HW), _row_spec(HW), _row_spec(OUT)]

    h4a0, h4b0, hs_0 = pl.pallas_call(
        _dense_in_body,
        grid=grid,
        in_specs=[_row_spec(INP), _row_spec(EMB), _full_spec((in0, 2 * HW + OUT))],
        out_specs=h4_specs,
        out_shape=h4_shapes,
    )(x, embed, W0)

    comp0p = jnp.repeat(comp0, MW, axis=1)
    comp1p = jnp.repeat(comp1, MW, axis=1)

    parts0 = _edge_pass(h4a0, h4b0, eidx, comp0p)

    h4a1, h4b1, hs_1 = pl.pallas_call(
        _fuse_mid_body,
        grid=grid,
        in_specs=[_parts_spec(), _row_spec(OUT), _full_spec((EMB, 2 * HW + OUT))],
        out_specs=h4_specs,
        out_shape=h4_shapes,
    )(parts0, hs_0, W1)

    parts1 = _edge_pass(h4a1, h4b1, eidx, comp1p)

    out = pl.pallas_call(
        _fuse_out_body,
        grid=grid,
        in_specs=[_parts_spec(), _row_spec(OUT)],
        out_specs=_row_spec(OUT),
        out_shape=jax.ShapeDtypeStruct((N, OUT), jnp.float32),
    )(parts1, hs_1)

    return out


# speedup vs baseline: 2.2115x; 1.1896x over previous
import functools

import jax
import jax.numpy as jnp
from jax import lax
from jax.experimental import pallas as pl
from jax.experimental.pallas import tpu as pltpu
from jax.experimental.pallas import tpu_sc as plsc

N = 50000
E = 800000
INP = 32
EMB = 32
NB = 4
NR = 64
OUT = 32
H4W = NB * OUT

NC = 2
NS = 16
CH = 128
HW = NB * 16
MW = 16
NCH = 392
EP = NS * NCH * CH
SUP = 28
NSUP = NCH // SUP
NPAIR = SUP // 2
STRIPE = N // NS
ZR = STRIPE // 5


def _edge_body(h4a, h4b, eidx, comp, out, ibuf, rows0, rows1, cf0, cf1,
               msg0, msg1, zbuf, agg, compS, gsem0, gsem1, csem0, csem1,
               ssem0, ssem1):
    cid = lax.axis_index("c")
    sid = lax.axis_index("s")
    r0 = sid * STRIPE

    @pl.when(sid == 0)
    def _stage_comp():
        pltpu.sync_copy(comp, compS)

    zeros16 = jnp.zeros((16,), jnp.float32)

    def zb(i, carry):
        zbuf[i, pl.ds(0, 16)] = zeros16
        return carry

    lax.fori_loop(0, ZR, zb, 0)
    for j in range(STRIPE // ZR):
        pltpu.sync_copy(zbuf, agg.at[pl.ds(r0 + j * ZR, ZR)])
    plsc.subcore_barrier()

    def make_compute(rows_b, cf_b, msg_b):
        def compute(k):
            del k

            def load(e):
                return [(cf_b[e, pl.ds(16 * b, 16)], rows_b[e, pl.ds(16 * b, 16)])
                        for b in range(NB)]

            def group(g, carry):
                e0 = g * 16
                prev = load(e0)
                for j in range(16):
                    cur = load(e0 + j + 1) if j < 15 else None
                    p = [c * r for c, r in prev]
                    msg_b[e0 + j, pl.ds(0, 16)] = (p[0] + p[1]) + (p[2] + p[3])
                    prev = cur
                return carry

            lax.fori_loop(0, CH // 16, group, 0)
        return compute

    def run_edges(h4):
        compute0 = make_compute(rows0, cf0, msg0)
        compute1 = make_compute(rows1, cf1, msg1)

        def g_start(k, rows_b, cf_b, gsem, csem):
            pltpu.async_copy(h4.at[ibuf.at[k, 0]], rows_b, gsem)
            pltpu.async_copy(compS.at[ibuf.at[k, 2]], cf_b, csem)

        def g_wait(rows_b, cf_b, gsem, csem):
            pltpu.make_async_copy(h4.at[ibuf.at[0, 0]], rows_b, gsem).wait()
            pltpu.make_async_copy(compS.at[ibuf.at[0, 2]], cf_b, csem).wait()

        def s_start(k, msg_b, sem):
            pltpu.async_copy(msg_b, agg.at[ibuf.at[k, 1]], sem, add=True)

        def s_wait(msg_b, sem):
            pltpu.make_async_copy(msg_b, agg.at[ibuf.at[0, 1]], sem).wait()

        def pair(j, carry):
            c0 = 2 * j
            g_wait(rows0, cf0, gsem0, csem0)

            @pl.when(j >= 1)
            def _():
                s_wait(msg0, ssem0)

            g_start(c0 + 1, rows1, cf1, gsem1, csem1)
            compute0(c0)
            s_start(c0, msg0, ssem0)

            g_wait(rows1, cf1, gsem1, csem1)

            @pl.when(j >= 1)
            def _():
                s_wait(msg1, ssem1)

            @pl.when(j < NPAIR - 1)
            def _():
                g_start(c0 + 2, rows0, cf0, gsem0, csem0)

            compute1(c0 + 1)
            s_start(c0 + 1, msg1, ssem1)
            return carry

        def super_body(s, carry):
            pltpu.sync_copy(eidx.at[sid, pl.ds(s * SUP, SUP)], ibuf)
            g_start(0, rows0, cf0, gsem0, csem0)
            lax.fori_loop(0, NPAIR, pair, 0)
            s_wait(msg0, ssem0)
            s_wait(msg1, ssem1)
            return carry

        lax.fori_loop(0, NSUP, super_body, 0)

    @pl.when(cid == 0)
    def _edges_a():
        run_edges(h4a)

    @pl.when(cid == 1)
    def _edges_b():
        run_edges(h4b)

    plsc.subcore_barrier()
    pltpu.sync_copy(agg.at[pl.ds(r0, STRIPE)], out.at[cid, pl.ds(r0, STRIPE)])


_edge_pass = functools.partial(
    pl.kernel,
    out_type=jax.ShapeDtypeStruct((NC, N, MW), jnp.float32),
    mesh=plsc.VectorSubcoreMesh(core_axis_name="c", subcore_axis_name="s"),
    compiler_params=pltpu.CompilerParams(use_tc_tiling_on_sc=False),
    scratch_types=[
        pltpu.VMEM((SUP, 3, CH), jnp.int32),
        pltpu.VMEM((CH, HW), jnp.float32),
        pltpu.VMEM((CH, HW), jnp.float32),
        pltpu.VMEM((CH, HW), jnp.float32),
        pltpu.VMEM((CH, HW), jnp.float32),
        pltpu.VMEM((CH, MW), jnp.float32),
        pltpu.VMEM((CH, MW), jnp.float32),
        pltpu.VMEM((ZR, MW), jnp.float32),
        pltpu.VMEM_SHARED((N + 16, MW), jnp.float32),
        pltpu.VMEM_SHARED((NR, HW), jnp.float32),
        pltpu.SemaphoreType.DMA,
        pltpu.SemaphoreType.DMA,
        pltpu.SemaphoreType.DMA,
        pltpu.SemaphoreType.DMA,
        pltpu.SemaphoreType.DMA,
        pltpu.SemaphoreType.DMA,
    ],
)(_edge_body)


BN = 1000


def _dense_in_body(x_ref, emb_ref, w_ref, h4a_ref, h4b_ref, hs_ref):
    h = jnp.concatenate([x_ref[...], emb_ref[...]], axis=1)
    o = jnp.dot(h, w_ref[...], preferred_element_type=jnp.float32)
    h4a_ref[...] = o[:, :HW]
    h4b_ref[...] = o[:, HW:2 * HW]
    hs_ref[...] = o[:, 2 * HW:]


def _fuse_mid_body(parts_ref, hs_ref, w_ref, h4a_ref, h4b_ref, hs1_ref):
    agg = jnp.concatenate([parts_ref[0], parts_ref[1]], axis=1)
    h = jnp.maximum(agg + hs_ref[...], 0.0)
    o = jnp.dot(h, w_ref[...], preferred_element_type=jnp.float32)
    h4a_ref[...] = o[:, :HW]
    h4b_ref[...] = o[:, HW:2 * HW]
    hs1_ref[...] = o[:, 2 * HW:]


def _fuse_out_body(parts_ref, hs_ref, out_ref):
    agg = jnp.concatenate([parts_ref[0], parts_ref[1]], axis=1)
    out_ref[...] = jnp.maximum(agg + hs_ref[...], 0.0)


def _row_spec(width):
    return pl.BlockSpec((BN, width), lambda i: (i, 0))


def _parts_spec():
    return pl.BlockSpec((NC, BN, MW), lambda i: (0, i, 0))


def _full_spec(shape):
    return pl.BlockSpec(shape, lambda i: tuple(0 for _ in shape))


def kernel(x, edge_index, edge_type, embed, basis0, comp0, Wself0,
           basis1, comp1, Wself1):
    in0 = INP + EMB

    pad = EP - E
    srcp = jnp.concatenate([edge_index[0], jnp.zeros((pad,), jnp.int32)])
    dstp = jnp.concatenate([edge_index[1], jnp.full((pad,), N, jnp.int32)])
    etp = jnp.concatenate([edge_type, jnp.zeros((pad,), jnp.int32)])
    eidx = (jnp.stack([srcp, dstp, etp], axis=0)
            .reshape(3, NS, NCH, CH).transpose(1, 2, 0, 3))

    def _wcat(basis, wself, kdim):
        wa = basis[:, :, :MW].transpose(1, 0, 2).reshape(kdim, HW)
        wb = basis[:, :, MW:].transpose(1, 0, 2).reshape(kdim, HW)
        return jnp.concatenate([wa, wb, wself], axis=1)

    W0 = _wcat(basis0, Wself0, in0)
    W1 = _wcat(basis1, Wself1, EMB)

    grid = (N // BN,)
    h4_shapes = [jax.ShapeDtypeStruct((N, HW), jnp.float32),
                 jax.ShapeDtypeStruct((N, HW), jnp.float32),
                 jax.ShapeDtypeStruct((N, OUT), jnp.float32)]
    h4_specs = [_row_spec(HW), _row_spec(HW), _row_spec(OUT)]

    h4a0, h4b0, hs_0 = pl.pallas_call(
        _dense_in_body,
        grid=grid,
        in_specs=[_row_spec(INP), _row_spec(EMB), _full_spec((in0, 2 * HW + OUT))],
        out_specs=h4_specs,
        out_shape=h4_shapes,
    )(x, embed, W0)

    comp0p = jnp.repeat(comp0, MW, axis=1)
    comp1p = jnp.repeat(comp1, MW, axis=1)

    parts0 = _edge_pass(h4a0, h4b0, eidx, comp0p)

    h4a1, h4b1, hs_1 = pl.pallas_call(
        _fuse_mid_body,
        grid=grid,
        in_specs=[_parts_spec(), _row_spec(OUT), _full_spec((EMB, 2 * HW + OUT))],
        out_specs=h4_specs,
        out_shape=h4_shapes,
    )(parts0, hs_0, W1)

    parts1 = _edge_pass(h4a1, h4b1, eidx, comp1p)

    out = pl.pallas_call(
        _fuse_out_body,
        grid=grid,
        in_specs=[_parts_spec(), _row_spec(OUT)],
        out_specs=_row_spec(OUT),
        out_shape=jax.ShapeDtypeStruct((N, OUT), jnp.float32),
    )(parts1, hs_1)

    return out


# speedup vs baseline: 2.2707x; 1.0268x over previous
import functools

import jax
import jax.numpy as jnp
from jax import lax
from jax.experimental import pallas as pl
from jax.experimental.pallas import tpu as pltpu
from jax.experimental.pallas import tpu_sc as plsc

N = 50000
E = 800000
INP = 32
EMB = 32
NB = 4
NR = 64
OUT = 32
H4W = NB * OUT

NC = 2
NS = 16
CH = 128
HW = NB * 16
MW = 16
NCH = 392
EP = NS * NCH * CH
SUP = 28
NSUP = NCH // SUP
NPAIR = SUP // 2
STRIPE = N // NS
ZR = STRIPE // 5


def _edge_body(h4a, h4b, esrc, edst, eet, comp, out, ibs, ibd, ibe,
               rows0, rows1, cf0, cf1, msg0, msg1, zbuf, agg, compS,
               gsem0, gsem1, csem0, csem1, ssem0, ssem1):
    cid = lax.axis_index("c")
    sid = lax.axis_index("s")
    r0 = sid * STRIPE

    @pl.when(sid == 0)
    def _stage_comp():
        pltpu.sync_copy(comp, compS)

    zeros16 = jnp.zeros((16,), jnp.float32)

    def zb(i, carry):
        zbuf[i, pl.ds(0, 16)] = zeros16
        return carry

    lax.fori_loop(0, ZR, zb, 0)
    for j in range(STRIPE // ZR):
        pltpu.sync_copy(zbuf, agg.at[pl.ds(r0 + j * ZR, ZR)])
    plsc.subcore_barrier()

    def make_compute(rows_b, cf_b, msg_b):
        def compute(k):
            del k

            def load(e):
                return [(cf_b[e, pl.ds(16 * b, 16)], rows_b[e, pl.ds(16 * b, 16)])
                        for b in range(NB)]

            def group(g, carry):
                e0 = g * 16
                prev = load(e0)
                for j in range(16):
                    cur = load(e0 + j + 1) if j < 15 else None
                    p = [c * r for c, r in prev]
                    msg_b[e0 + j, pl.ds(0, 16)] = (p[0] + p[1]) + (p[2] + p[3])
                    prev = cur
                return carry

            lax.fori_loop(0, CH // 16, group, 0)
        return compute

    def run_edges(h4):
        compute0 = make_compute(rows0, cf0, msg0)
        compute1 = make_compute(rows1, cf1, msg1)

        def g_start(k, rows_b, cf_b, gsem, csem):
            pltpu.async_copy(h4.at[ibs.at[k]], rows_b, gsem)
            pltpu.async_copy(compS.at[ibe.at[k]], cf_b, csem)

        def g_wait(rows_b, cf_b, gsem, csem):
            pltpu.make_async_copy(h4.at[ibs.at[0]], rows_b, gsem).wait()
            pltpu.make_async_copy(compS.at[ibe.at[0]], cf_b, csem).wait()

        def s_start(k, msg_b, sem):
            pltpu.async_copy(msg_b, agg.at[ibd.at[k]], sem, add=True)

        def s_wait(msg_b, sem):
            pltpu.make_async_copy(msg_b, agg.at[ibd.at[0]], sem).wait()

        def pair(j, carry):
            c0 = 2 * j
            g_wait(rows0, cf0, gsem0, csem0)

            @pl.when(j >= 1)
            def _():
                s_wait(msg0, ssem0)

            g_start(c0 + 1, rows1, cf1, gsem1, csem1)
            compute0(c0)
            s_start(c0, msg0, ssem0)

            g_wait(rows1, cf1, gsem1, csem1)

            @pl.when(j >= 1)
            def _():
                s_wait(msg1, ssem1)

            @pl.when(j < NPAIR - 1)
            def _():
                g_start(c0 + 2, rows0, cf0, gsem0, csem0)

            compute1(c0 + 1)
            s_start(c0 + 1, msg1, ssem1)
            return carry

        def super_body(s, carry):
            pltpu.sync_copy(esrc.at[sid, pl.ds(s * SUP, SUP)], ibs)
            pltpu.sync_copy(edst.at[sid, pl.ds(s * SUP, SUP)], ibd)
            pltpu.sync_copy(eet.at[sid, pl.ds(s * SUP, SUP)], ibe)
            g_start(0, rows0, cf0, gsem0, csem0)
            lax.fori_loop(0, NPAIR, pair, 0)
            s_wait(msg0, ssem0)
            s_wait(msg1, ssem1)
            return carry

        lax.fori_loop(0, NSUP, super_body, 0)

    @pl.when(cid == 0)
    def _edges_a():
        run_edges(h4a)

    @pl.when(cid == 1)
    def _edges_b():
        run_edges(h4b)

    plsc.subcore_barrier()
    pltpu.sync_copy(agg.at[pl.ds(r0, STRIPE)], out.at[cid, pl.ds(r0, STRIPE)])


_edge_pass = functools.partial(
    pl.kernel,
    out_type=jax.ShapeDtypeStruct((NC, N, MW), jnp.float32),
    mesh=plsc.VectorSubcoreMesh(core_axis_name="c", subcore_axis_name="s"),
    compiler_params=pltpu.CompilerParams(use_tc_tiling_on_sc=False),
    scratch_types=[
        pltpu.VMEM((SUP, CH), jnp.int32),
        pltpu.VMEM((SUP, CH), jnp.int32),
        pltpu.VMEM((SUP, CH), jnp.int32),
        pltpu.VMEM((CH, HW), jnp.float32),
        pltpu.VMEM((CH, HW), jnp.float32),
        pltpu.VMEM((CH, HW), jnp.float32),
        pltpu.VMEM((CH, HW), jnp.float32),
        pltpu.VMEM((CH, MW), jnp.float32),
        pltpu.VMEM((CH, MW), jnp.float32),
        pltpu.VMEM((ZR, MW), jnp.float32),
        pltpu.VMEM_SHARED((N + 16, MW), jnp.float32),
        pltpu.VMEM_SHARED((NR, HW), jnp.float32),
        pltpu.SemaphoreType.DMA,
        pltpu.SemaphoreType.DMA,
        pltpu.SemaphoreType.DMA,
        pltpu.SemaphoreType.DMA,
        pltpu.SemaphoreType.DMA,
        pltpu.SemaphoreType.DMA,
    ],
)(_edge_body)


BN = 5000


def _dense_in_body(x_ref, emb_ref, w_ref, h4a_ref, h4b_ref, hs_ref):
    h = jnp.concatenate([x_ref[...], emb_ref[...]], axis=1)
    o = jnp.dot(h, w_ref[...], preferred_element_type=jnp.float32)
    h4a_ref[...] = o[:, :HW]
    h4b_ref[...] = o[:, HW:2 * HW]
    hs_ref[...] = o[:, 2 * HW:]


def _fuse_mid_body(parts_ref, hs_ref, w_ref, h4a_ref, h4b_ref, hs1_ref):
    agg = jnp.concatenate([parts_ref[0], parts_ref[1]], axis=1)
    h = jnp.maximum(agg + hs_ref[...], 0.0)
    o = jnp.dot(h, w_ref[...], preferred_element_type=jnp.float32)
    h4a_ref[...] = o[:, :HW]
    h4b_ref[...] = o[:, HW:2 * HW]
    hs1_ref[...] = o[:, 2 * HW:]


def _fuse_out_body(parts_ref, hs_ref, out_ref):
    agg = jnp.concatenate([parts_ref[0], parts_ref[1]], axis=1)
    out_ref[...] = jnp.maximum(agg + hs_ref[...], 0.0)


def _row_spec(width):
    return pl.BlockSpec((BN, width), lambda i: (i, 0))


def _parts_spec():
    return pl.BlockSpec((NC, BN, MW), lambda i: (0, i, 0))


def _full_spec(shape):
    return pl.BlockSpec(shape, lambda i: tuple(0 for _ in shape))


def kernel(x, edge_index, edge_type, embed, basis0, comp0, Wself0,
           basis1, comp1, Wself1):
    in0 = INP + EMB

    pad = EP - E
    srcp = jnp.concatenate([edge_index[0], jnp.zeros((pad,), jnp.int32)]
                           ).reshape(NS, NCH, CH)
    dstp = jnp.concatenate([edge_index[1], jnp.full((pad,), N, jnp.int32)]
                           ).reshape(NS, NCH, CH)
    etp = jnp.concatenate([edge_type, jnp.zeros((pad,), jnp.int32)]
                          ).reshape(NS, NCH, CH)

    def _wcat(basis, wself, kdim):
        wa = basis[:, :, :MW].transpose(1, 0, 2).reshape(kdim, HW)
        wb = basis[:, :, MW:].transpose(1, 0, 2).reshape(kdim, HW)
        return jnp.concatenate([wa, wb, wself], axis=1)

    W0 = _wcat(basis0, Wself0, in0)
    W1 = _wcat(basis1, Wself1, EMB)

    grid = (N // BN,)
    h4_shapes = [jax.ShapeDtypeStruct((N, HW), jnp.float32),
                 jax.ShapeDtypeStruct((N, HW), jnp.float32),
                 jax.ShapeDtypeStruct((N, OUT), jnp.float32)]
    h4_specs = [_row_spec(HW), _row_spec(HW), _row_spec(OUT)]

    h4a0, h4b0, hs_0 = pl.pallas_call(
        _dense_in_body,
        grid=grid,
        in_specs=[_row_spec(INP), _row_spec(EMB), _full_spec((in0, 2 * HW + OUT))],
        out_specs=h4_specs,
        out_shape=h4_shapes,
    )(x, embed, W0)

    comp0p = jnp.repeat(comp0, MW, axis=1)
    comp1p = jnp.repeat(comp1, MW, axis=1)

    parts0 = _edge_pass(h4a0, h4b0, srcp, dstp, etp, comp0p)

    h4a1, h4b1, hs_1 = pl.pallas_call(
        _fuse_mid_body,
        grid=grid,
        in_specs=[_parts_spec(), _row_spec(OUT), _full_spec((EMB, 2 * HW + OUT))],
        out_specs=h4_specs,
        out_shape=h4_shapes,
    )(parts0, hs_0, W1)

    parts1 = _edge_pass(h4a1, h4b1, srcp, dstp, etp, comp1p)

    out = pl.pallas_call(
        _fuse_out_body,
        grid=grid,
        in_specs=[_parts_spec(), _row_spec(OUT)],
        out_specs=_row_spec(OUT),
        out_shape=jax.ShapeDtypeStruct((N, OUT), jnp.float32),
    )(parts1, hs_1)

    return out


# speedup vs baseline: 2.3279x; 1.0252x over previous
import functools

import jax
import jax.numpy as jnp
from jax import lax
from jax.experimental import pallas as pl
from jax.experimental.pallas import tpu as pltpu
from jax.experimental.pallas import tpu_sc as plsc

N = 50000
E = 800000
INP = 32
EMB = 32
NB = 4
NR = 64
OUT = 32
H4W = NB * OUT

NC = 2
NS = 16
CH = 128
HW = NB * 16
MW = 16
NCH = 392
EP = NS * NCH * CH
SUP = 56
NSUP = NCH // SUP
NPAIR = SUP // 2
STRIPE = N // NS
ZR = STRIPE // 5


def _edge_body(h4a, h4b, esrc, edst, eet, comp, out, ibs, ibd, ibe,
               rows0, rows1, cf0, cf1, msg0, msg1, zbuf, agg, compS,
               gsem0, gsem1, csem0, csem1, ssem0, ssem1):
    cid = lax.axis_index("c")
    sid = lax.axis_index("s")
    r0 = sid * STRIPE

    @pl.when(sid == 0)
    def _stage_comp():
        pltpu.sync_copy(comp, compS)

    zeros16 = jnp.zeros((16,), jnp.float32)

    def zb(i, carry):
        zbuf[i, pl.ds(0, 16)] = zeros16
        return carry

    lax.fori_loop(0, ZR, zb, 0)
    for j in range(STRIPE // ZR):
        pltpu.sync_copy(zbuf, agg.at[pl.ds(r0 + j * ZR, ZR)])
    plsc.subcore_barrier()

    def make_compute(rows_b, cf_b, msg_b):
        def compute(k):
            del k

            def load(e):
                return [(cf_b[e, pl.ds(16 * b, 16)], rows_b[e, pl.ds(16 * b, 16)])
                        for b in range(NB)]

            def group(g, carry):
                e0 = g * 16
                prev = load(e0)
                for j in range(16):
                    cur = load(e0 + j + 1) if j < 15 else None
                    p = [c * r for c, r in prev]
                    msg_b[e0 + j, pl.ds(0, 16)] = (p[0] + p[1]) + (p[2] + p[3])
                    prev = cur
                return carry

            lax.fori_loop(0, CH // 16, group, 0)
        return compute

    def run_edges(h4):
        compute0 = make_compute(rows0, cf0, msg0)
        compute1 = make_compute(rows1, cf1, msg1)

        def g_start(k, rows_b, cf_b, gsem, csem):
            pltpu.async_copy(h4.at[ibs.at[k]], rows_b, gsem)
            pltpu.async_copy(compS.at[ibe.at[k]], cf_b, csem)

        def g_wait(rows_b, cf_b, gsem, csem):
            pltpu.make_async_copy(h4.at[ibs.at[0]], rows_b, gsem).wait()
            pltpu.make_async_copy(compS.at[ibe.at[0]], cf_b, csem).wait()

        def s_start(k, msg_b, sem):
            pltpu.async_copy(msg_b, agg.at[ibd.at[k]], sem, add=True)

        def s_wait(msg_b, sem):
            pltpu.make_async_copy(msg_b, agg.at[ibd.at[0]], sem).wait()

        def pair(j, carry):
            c0 = 2 * j
            g_wait(rows0, cf0, gsem0, csem0)

            @pl.when(j >= 1)
            def _():
                s_wait(msg0, ssem0)

            g_start(c0 + 1, rows1, cf1, gsem1, csem1)
            compute0(c0)
            s_start(c0, msg0, ssem0)

            g_wait(rows1, cf1, gsem1, csem1)

            @pl.when(j >= 1)
            def _():
                s_wait(msg1, ssem1)

            @pl.when(j < NPAIR - 1)
            def _():
                g_start(c0 + 2, rows0, cf0, gsem0, csem0)

            compute1(c0 + 1)
            s_start(c0 + 1, msg1, ssem1)
            return carry

        def super_body(s, carry):
            pltpu.sync_copy(esrc.at[sid, pl.ds(s * SUP, SUP)], ibs)
            pltpu.sync_copy(edst.at[sid, pl.ds(s * SUP, SUP)], ibd)
            pltpu.sync_copy(eet.at[sid, pl.ds(s * SUP, SUP)], ibe)
            g_start(0, rows0, cf0, gsem0, csem0)
            lax.fori_loop(0, NPAIR, pair, 0)
            s_wait(msg0, ssem0)
            s_wait(msg1, ssem1)
            return carry

        lax.fori_loop(0, NSUP, super_body, 0)

    @pl.when(cid == 0)
    def _edges_a():
        run_edges(h4a)

    @pl.when(cid == 1)
    def _edges_b():
        run_edges(h4b)

    plsc.subcore_barrier()
    pltpu.sync_copy(agg.at[pl.ds(r0, STRIPE)], out.at[cid, pl.ds(r0, STRIPE)])


_edge_pass = functools.partial(
    pl.kernel,
    out_type=jax.ShapeDtypeStruct((NC, N, MW), jnp.float32),
    mesh=plsc.VectorSubcoreMesh(core_axis_name="c", subcore_axis_name="s"),
    compiler_params=pltpu.CompilerParams(use_tc_tiling_on_sc=False),
    scratch_types=[
        pltpu.VMEM((SUP, CH), jnp.int32),
        pltpu.VMEM((SUP, CH), jnp.int32),
        pltpu.VMEM((SUP, CH), jnp.int32),
        pltpu.VMEM((CH, HW), jnp.float32),
        pltpu.VMEM((CH, HW), jnp.float32),
        pltpu.VMEM((CH, HW), jnp.float32),
        pltpu.VMEM((CH, HW), jnp.float32),
        pltpu.VMEM((CH, MW), jnp.float32),
        pltpu.VMEM((CH, MW), jnp.float32),
        pltpu.VMEM((ZR, MW), jnp.float32),
        pltpu.VMEM_SHARED((N + 16, MW), jnp.float32),
        pltpu.VMEM_SHARED((NR, HW), jnp.float32),
        pltpu.SemaphoreType.DMA,
        pltpu.SemaphoreType.DMA,
        pltpu.SemaphoreType.DMA,
        pltpu.SemaphoreType.DMA,
        pltpu.SemaphoreType.DMA,
        pltpu.SemaphoreType.DMA,
    ],
)(_edge_body)


BN = 5000


def _dense_in_body(x_ref, emb_ref, w_ref, h4a_ref, h4b_ref, hs_ref):
    h = jnp.concatenate([x_ref[...], emb_ref[...]], axis=1)
    o = jnp.dot(h, w_ref[...], preferred_element_type=jnp.float32)
    h4a_ref[...] = o[:, :HW]
    h4b_ref[...] = o[:, HW:2 * HW]
    hs_ref[...] = o[:, 2 * HW:]


def _fuse_mid_body(parts_ref, hs_ref, w_ref, h4a_ref, h4b_ref, hs1_ref):
    agg = jnp.concatenate([parts_ref[0], parts_ref[1]], axis=1)
    h = jnp.maximum(agg + hs_ref[...], 0.0)
    o = jnp.dot(h, w_ref[...], preferred_element_type=jnp.float32)
    h4a_ref[...] = o[:, :HW]
    h4b_ref[...] = o[:, HW:2 * HW]
    hs1_ref[...] = o[:, 2 * HW:]


def _fuse_out_body(parts_ref, hs_ref, out_ref):
    agg = jnp.concatenate([parts_ref[0], parts_ref[1]], axis=1)
    out_ref[...] = jnp.maximum(agg + hs_ref[...], 0.0)


def _row_spec(width):
    return pl.BlockSpec((BN, width), lambda i: (i, 0))


def _parts_spec():
    return pl.BlockSpec((NC, BN, MW), lambda i: (0, i, 0))


def _full_spec(shape):
    return pl.BlockSpec(shape, lambda i: tuple(0 for _ in shape))


def kernel(x, edge_index, edge_type, embed, basis0, comp0, Wself0,
           basis1, comp1, Wself1):
    in0 = INP + EMB

    pad = EP - E
    srcp = jnp.concatenate([edge_index[0], jnp.zeros((pad,), jnp.int32)]
                           ).reshape(NS, NCH, CH)
    dstp = jnp.concatenate([edge_index[1], jnp.full((pad,), N, jnp.int32)]
                           ).reshape(NS, NCH, CH)
    etp = jnp.concatenate([edge_type, jnp.zeros((pad,), jnp.int32)]
                          ).reshape(NS, NCH, CH)

    def _wcat(basis, wself, kdim):
        wa = basis[:, :, :MW].transpose(1, 0, 2).reshape(kdim, HW)
        wb = basis[:, :, MW:].transpose(1, 0, 2).reshape(kdim, HW)
        return jnp.concatenate([wa, wb, wself], axis=1)

    W0 = _wcat(basis0, Wself0, in0)
    W1 = _wcat(basis1, Wself1, EMB)

    grid = (N // BN,)
    h4_shapes = [jax.ShapeDtypeStruct((N, HW), jnp.float32),
                 jax.ShapeDtypeStruct((N, HW), jnp.float32),
                 jax.ShapeDtypeStruct((N, OUT), jnp.float32)]
    h4_specs = [_row_spec(HW), _row_spec(HW), _row_spec(OUT)]

    h4a0, h4b0, hs_0 = pl.pallas_call(
        _dense_in_body,
        grid=grid,
        in_specs=[_row_spec(INP), _row_spec(EMB), _full_spec((in0, 2 * HW + OUT))],
        out_specs=h4_specs,
        out_shape=h4_shapes,
    )(x, embed, W0)

    comp0p = jnp.repeat(comp0, MW, axis=1)
    comp1p = jnp.repeat(comp1, MW, axis=1)

    parts0 = _edge_pass(h4a0, h4b0, srcp, dstp, etp, comp0p)

    h4a1, h4b1, hs_1 = pl.pallas_call(
        _fuse_mid_body,
        grid=grid,
        in_specs=[_parts_spec(), _row_spec(OUT), _full_spec((EMB, 2 * HW + OUT))],
        out_specs=h4_specs,
        out_shape=h4_shapes,
    )(parts0, hs_0, W1)

    parts1 = _edge_pass(h4a1, h4b1, srcp, dstp, etp, comp1p)

    out = pl.pallas_call(
        _fuse_out_body,
        grid=grid,
        in_specs=[_parts_spec(), _row_spec(OUT)],
        out_specs=_row_spec(OUT),
        out_shape=jax.ShapeDtypeStruct((N, OUT), jnp.float32),
    )(parts1, hs_1)

    return out
